# baseline (speedup 1.0000x reference)
import functools

import jax
import jax.numpy as jnp
import numpy as np
from jax import lax
from jax.experimental import pallas as pl
from jax.experimental.pallas import tpu as pltpu
from jax.experimental.pallas import tpu_sc as plsc

_N = 32
_F = 16
_OBS = 16
_HID = 64
_ACT = 2
_KP = 1.0
_MMAX = 1.0
_BB = 512


def _sc_gather(adj_c, nobs_c, agent_id_flat):
    B = agent_id_flat.shape[0]
    info = plsc.get_sparse_core_info()
    nc, ns, L = info.num_cores, info.num_subcores, info.num_lanes
    nw = nc * ns
    bpw = B // nw
    mesh = plsc.VectorSubcoreMesh(core_axis_name="c", subcore_axis_name="s")

    @functools.partial(
        pl.kernel,
        mesh=mesh,
        out_type=[
            jax.ShapeDtypeStruct((B, 128), jnp.float32),
            jax.ShapeDtypeStruct((B, 128), jnp.float32),
        ],
        scratch_types=[
            pltpu.VMEM((bpw,), jnp.int32),
            pltpu.VMEM((bpw,), jnp.int32),
            pltpu.VMEM((bpw,), jnp.int32),
            pltpu.VMEM((bpw, 128), jnp.float32),
            pltpu.VMEM((bpw, 128), jnp.float32),
            pltpu.SemaphoreType.DMA,
            pltpu.SemaphoreType.DMA,
        ],
    )
    def k(adj_hbm, nobs_hbm, aid_hbm, adjrow_out, nobsag_out,
          aid_v, idxa_v, idxn_v, adjrow_v, nobsag_v, sem_a, sem_n):
        wid = lax.axis_index("s") * nc + lax.axis_index("c")
        base = wid * bpw
        pltpu.sync_copy(aid_hbm.at[pl.ds(base, bpw)], aid_v)
        lane = lax.iota(jnp.int32, L)
        for i in range(bpw // L):
            ids = aid_v[pl.ds(i * L, L)]
            samp = lane + (base + i * L)
            idxa_v[pl.ds(i * L, L)] = samp * 8 + lax.shift_right_logical(ids, 2)
            idxn_v[pl.ds(i * L, L)] = samp * 4 + lax.shift_right_logical(ids, 3)
        ca = pltpu.async_copy(adj_hbm.at[idxa_v], adjrow_v, sem_a)
        cn = pltpu.async_copy(nobs_hbm.at[idxn_v], nobsag_v, sem_n)
        ca.wait()
        cn.wait()
        pltpu.sync_copy(adjrow_v, adjrow_out.at[pl.ds(base, bpw)])
        pltpu.sync_copy(nobsag_v, nobsag_out.at[pl.ds(base, bpw)])

    return k(adj_c, nobs_c, agent_id_flat)


def _tc_body(obs_r, nobs_r, aid_r, chka_r, chkn_r, dist_r, sre_r, sim_r, msk_r,
             W1_r, W2_r, Wd_r, Wd2_r, Wm1_r, bm1_r, Wm2_r, bm2_r, Wa_r, ba_r,
             lstd_r, lre_r, lim_r, Bre_r, Bim_r, Cre_r, Cim_r, D_r, Wsm_r,
             Wso_r, bso_r, act_out, alp_out, nsre_out, nsim_out):
    f32 = jnp.float32

    def dot(a, b):
        return lax.dot_general(a, b, (((1,), (0,)), ((), ())),
                               preferred_element_type=f32)

    def bf(a):
        return a.astype(jnp.bfloat16).astype(f32)

    aid = aid_r[...]
    chka = chka_r[...]
    k4 = aid & 3
    arow = jnp.where(k4 == 0, chka[:, 0:_N], 0.0)
    arow += jnp.where(k4 == 1, chka[:, _N:2 * _N], 0.0)
    arow += jnp.where(k4 == 2, chka[:, 2 * _N:3 * _N], 0.0)
    arow += jnp.where(k4 == 3, chka[:, 3 * _N:], 0.0)
    chkn = chkn_r[...]
    k8 = aid & 7
    nobs_ag = jnp.where(k8 == 0, chkn[:, 0:_F], 0.0)
    for kk in range(1, 8):
        nobs_ag += jnp.where(k8 == kk, chkn[:, kk * _F:(kk + 1) * _F], 0.0)
    nobs = nobs_r[...]
    W1 = W1_r[...]
    h = jax.nn.relu(dot(nobs.reshape(_BB * _N, _F), W1))
    h3 = h.reshape(_BB, _N, _HID)
    arow = bf(arow)
    agg = jnp.sum(bf(h3) * arow[:, :, None], axis=1)
    h_ag = jax.nn.relu(dot(nobs_ag, W1))
    W2 = W2_r[...]
    nbd = jax.nn.relu(dot(h_ag, W2[:_HID]) + dot(agg, W2[_HID:]))
    obs = obs_r[...]
    Wm1 = Wm1_r[...]
    x = jax.nn.relu(dot(obs, Wm1[:_OBS]) + dot(nbd, Wm1[_OBS:]) + bm1_r[...])
    actor = jax.nn.relu(dot(x, Wm2_r[...]) + bm2_r[...])
    mean = dot(actor, Wa_r[...]) + ba_r[...]
    u_gnn = jnp.tanh(mean)

    dn = jax.nn.relu(dot(dist_r[...], Wd_r[...]))
    rs = jnp.sum(arow, axis=1, keepdims=True)
    Wd2 = Wd2_r[...]
    mag_g = jax.nn.relu(dot(dn, Wd2[:_HID]) + dot(rs * bf(dn), Wd2[_HID:]))

    reset = msk_r[...] == 0.0
    s_re = jnp.where(reset, 0.0, sre_r[...])
    s_im = jnp.where(reset, 0.0, sim_r[...])
    lre = lre_r[...]
    lim = lim_r[...]
    ns_re = lre * s_re - lim * s_im + dot(mag_g, Bre_r[...])
    ns_im = lre * s_im + lim * s_re + dot(mag_g, Bim_r[...])
    y_lru = dot(ns_re, Cre_r[...]) - dot(ns_im, Cim_r[...]) + dot(mag_g, D_r[...])
    ssm_raw = dot(jax.nn.relu(dot(y_lru, Wsm_r[...])), Wso_r[...]) + bso_r[...]
    magnitude = jnp.clip(jax.nn.relu(ssm_raw), 1e-6, _MMAX)

    actions = obs[:, 4:6] * _KP + magnitude * u_gnn
    lp = jnp.sum(-lstd_r[...] - 0.5 * np.float32(np.log(2.0 * np.pi)))
    ljt = jnp.sum(jnp.log(1.0 - u_gnn * u_gnn + 1e-8), axis=-1, keepdims=True)
    ljm = jnp.log(magnitude + 1e-8) * float(_ACT)
    act_out[...] = actions
    alp_out[...] = lp - ljm - ljt
    nsre_out[...] = ns_re
    nsim_out[...] = ns_im


def _tc_call(obs, node_obs, aid2, chka, chkn, dist, s_re, s_im, masks,
             W1, W2, Wd, Wd2, Wm1, bm1, Wm2, bm2, Wa, ba, lstd, lre, lim,
             Bre, Bim, Cre, Cim, D, Wsm, Wso, bso):
    B = obs.shape[0]
    grid = (B // _BB,)

    def bspec(shape):
        return pl.BlockSpec((_BB,) + shape[1:],
                            lambda i: (i,) + (0,) * (len(shape) - 1))

    def wspec(shape):
        return pl.BlockSpec(shape, lambda i: (0,) * len(shape))

    batch_args = [obs, node_obs, aid2, chka, chkn, dist, s_re, s_im, masks]
    weight_args = [W1, W2, Wd, Wd2, Wm1, bm1, Wm2, bm2, Wa, ba, lstd, lre,
                   lim, Bre, Bim, Cre, Cim, D, Wsm, Wso, bso]
    in_specs = [bspec(a.shape) for a in batch_args] + \
               [wspec(a.shape) for a in weight_args]
    out_shapes = [
        jax.ShapeDtypeStruct((B, _ACT), jnp.float32),
        jax.ShapeDtypeStruct((B, 1), jnp.float32),
        jax.ShapeDtypeStruct((B, _HID), jnp.float32),
        jax.ShapeDtypeStruct((B, _HID), jnp.float32),
    ]
    out_specs = [bspec(s.shape) for s in out_shapes]
    return pl.pallas_call(
        _tc_body,
        grid=grid,
        in_specs=in_specs,
        out_specs=out_specs,
        out_shape=out_shapes,
    )(*batch_args, *weight_args)


def kernel(obs, node_obs, adj, agent_id, rnn_states, ssm_state_re,
           ssm_state_im, disturbances, masks, W1, W2, Wd, Wd2, Wm1, bm1, Wm2,
           bm2, Wa, ba, log_std, lam_re, lam_im, B_re, B_im, C_re, C_im, D,
           Wsm, Wso, bso):
    B = obs.shape[0]
    aid_flat = agent_id.reshape(B).astype(jnp.int32)
    chka, chkn = _sc_gather(
        adj.reshape(B * _N * _N // 128, 128),
        node_obs.reshape(B * _N * _F // 128, 128),
        aid_flat,
    )
    actions, alp, ns_re, ns_im = _tc_call(
        obs, node_obs, aid_flat.reshape(B, 1), chka, chkn, disturbances,
        ssm_state_re, ssm_state_im, masks,
        W1, W2, Wd, Wd2, Wm1, bm1.reshape(1, _HID), Wm2,
        bm2.reshape(1, _HID), Wa,
        ba.reshape(1, _ACT), log_std.reshape(1, _ACT),
        lam_re.reshape(1, _HID), lam_im.reshape(1, _HID),
        B_re, B_im, C_re, C_im, D, Wsm, Wso, bso.reshape(1, 1),
    )
    return (actions, alp, rnn_states, ns_re, ns_im)


# speedup vs baseline: 3.0327x; 3.0327x over previous
import functools

import jax
import jax.numpy as jnp
import numpy as np
from jax import lax
from jax.experimental import pallas as pl
from jax.experimental.pallas import tpu as pltpu
from jax.experimental.pallas import tpu_sc as plsc

_N = 32
_F = 16
_OBS = 16
_HID = 64
_ACT = 2
_KP = 1.0
_MMAX = 1.0
_BB = 512


def _sc_gather_rows(adj_flat, agent_id_flat):
    B = agent_id_flat.shape[0]
    info = plsc.get_sparse_core_info()
    nc, ns, L = info.num_cores, info.num_subcores, info.num_lanes
    nw = nc * ns
    bpw = B // nw
    nb = B * _N
    mesh = plsc.VectorSubcoreMesh(core_axis_name="c", subcore_axis_name="s")

    @functools.partial(
        pl.kernel,
        mesh=mesh,
        out_type=jax.ShapeDtypeStruct((_N, B), jnp.float32),
        scratch_types=[
            pltpu.VMEM((bpw,), jnp.int32),
            pltpu.VMEM((_N, bpw), jnp.int32),
            pltpu.VMEM((_N, bpw), jnp.float32),
            pltpu.SemaphoreType.DMA,
        ],
    )
    def k(adj_hbm, aid_hbm, arow_out, aid_v, idx_v, row_v, sem):
        wid = lax.axis_index("s") * nc + lax.axis_index("c")
        base = wid * bpw
        pltpu.sync_copy(aid_hbm.at[pl.ds(base, bpw)], aid_v)
        lane = lax.iota(jnp.int32, L)
        for j in range(_N):
            for i in range(bpw // L):
                ids = aid_v[pl.ds(i * L, L)]
                idx_v[j, pl.ds(i * L, L)] = (
                    ids * nb + (j * B + base + i * L) + lane)
        copies = [
            pltpu.async_copy(adj_hbm.at[idx_v.at[j]], row_v.at[j], sem)
            for j in range(_N)
        ]
        for c in copies:
            c.wait()
        for j in range(_N):
            pltpu.sync_copy(row_v.at[j], arow_out.at[j, pl.ds(base, bpw)])

    return k(adj_flat, agent_id_flat)


def _tc_body(obs_r, nobs_r, aid_r, arow_r, dist_r, sre_r, sim_r, msk_r,
             W1_r, W2_r, Wd_r, Wd2_r, Wm1_r, bm1_r, Wm2_r, bm2_r, Wa_r, ba_r,
             lstd_r, lre_r, lim_r, Bre_r, Bim_r, Cre_r, Cim_r, D_r, Wsm_r,
             Wso_r, bso_r, act_out, alp_out, nsre_out, nsim_out):
    f32 = jnp.float32

    def dotT(w, x):
        return lax.dot_general(w, x, (((0,), (0,)), ((), ())),
                               preferred_element_type=f32)

    def bf(a):
        return a.astype(jnp.bfloat16).astype(f32)

    def col(v_r):
        return v_r[...].T

    aid = aid_r[...]
    oh = (lax.broadcasted_iota(jnp.int32, (_N, 1), 0) == aid).astype(f32)
    arow = arow_r[...]
    arow_q = bf(arow)
    rs = jnp.sum(arow_q, axis=0, keepdims=True)

    W1 = W1_r[...]
    agg = jnp.zeros((_HID, _BB), f32)
    hag = jnp.zeros((_HID, _BB), f32)
    for j in range(_N):
        h_j = jax.nn.relu(dotT(W1, nobs_r[j]))
        agg += arow_q[j:j + 1, :] * bf(h_j)
        hag += oh[j:j + 1, :] * h_j

    W2 = W2_r[...]
    nbd = jax.nn.relu(dotT(W2[:_HID], hag) + dotT(W2[_HID:], agg))
    obs = obs_r[...]
    Wm1 = Wm1_r[...]
    x = jax.nn.relu(dotT(Wm1[:_OBS], obs) + dotT(Wm1[_OBS:], nbd)
                    + col(bm1_r))
    actor = jax.nn.relu(dotT(Wm2_r[...], x) + col(bm2_r))
    mean = dotT(Wa_r[...], actor) + col(ba_r)
    u_gnn = jnp.tanh(mean)

    dn = jax.nn.relu(dotT(Wd_r[...], dist_r[...]))
    Wd2 = Wd2_r[...]
    mag_g = jax.nn.relu(dotT(Wd2[:_HID], dn) + dotT(Wd2[_HID:], rs * bf(dn)))

    reset = msk_r[...] == 0.0
    s_re = jnp.where(reset, 0.0, sre_r[...])
    s_im = jnp.where(reset, 0.0, sim_r[...])
    lre = col(lre_r)
    lim = col(lim_r)
    ns_re = lre * s_re - lim * s_im + dotT(Bre_r[...], mag_g)
    ns_im = lre * s_im + lim * s_re + dotT(Bim_r[...], mag_g)
    y_lru = (dotT(Cre_r[...], ns_re) - dotT(Cim_r[...], ns_im)
             + dotT(D_r[...], mag_g))
    ssm_raw = dotT(Wso_r[...], jax.nn.relu(dotT(Wsm_r[...], y_lru))) \
        + col(bso_r)
    magnitude = jnp.clip(jax.nn.relu(ssm_raw), 1e-6, _MMAX)

    actions = obs[4:6, :] * _KP + magnitude * u_gnn
    lp = jnp.sum(-lstd_r[...] - 0.5 * np.float32(np.log(2.0 * np.pi)))
    ljt = jnp.sum(jnp.log(1.0 - u_gnn * u_gnn + 1e-8), axis=0, keepdims=True)
    ljm = jnp.log(magnitude + 1e-8) * float(_ACT)
    act_out[...] = actions
    alp_out[...] = lp - ljm - ljt
    nsre_out[...] = ns_re
    nsim_out[...] = ns_im


def _tc_call(obsT, nobsT, aidT, arowT, distT, sreT, simT, mskT,
             W1, W2, Wd, Wd2, Wm1, bm1, Wm2, bm2, Wa, ba, lstd, lre, lim,
             Bre, Bim, Cre, Cim, D, Wsm, Wso, bso):
    B = obsT.shape[1]
    grid = (B // _BB,)

    def bspec(shape):
        return pl.BlockSpec(shape[:-1] + (_BB,),
                            lambda i: (0,) * (len(shape) - 1) + (i,))

    def wspec(shape):
        return pl.BlockSpec(shape, lambda i: (0,) * len(shape))

    batch_args = [obsT, nobsT, aidT, arowT, distT, sreT, simT, mskT]
    weight_args = [W1, W2, Wd, Wd2, Wm1, bm1, Wm2, bm2, Wa, ba, lstd, lre,
                   lim, Bre, Bim, Cre, Cim, D, Wsm, Wso, bso]
    in_specs = [bspec(a.shape) for a in batch_args] + \
               [wspec(a.shape) for a in weight_args]
    out_shapes = [
        jax.ShapeDtypeStruct((_ACT, B), jnp.float32),
        jax.ShapeDtypeStruct((1, B), jnp.float32),
        jax.ShapeDtypeStruct((_HID, B), jnp.float32),
        jax.ShapeDtypeStruct((_HID, B), jnp.float32),
    ]
    out_specs = [bspec(s.shape) for s in out_shapes]
    return pl.pallas_call(
        _tc_body,
        grid=grid,
        in_specs=in_specs,
        out_specs=out_specs,
        out_shape=out_shapes,
    )(*batch_args, *weight_args)


def kernel(obs, node_obs, adj, agent_id, rnn_states, ssm_state_re,
           ssm_state_im, disturbances, masks, W1, W2, Wd, Wd2, Wm1, bm1, Wm2,
           bm2, Wa, ba, log_std, lam_re, lam_im, B_re, B_im, C_re, C_im, D,
           Wsm, Wso, bso):
    B = obs.shape[0]
    aid_flat = agent_id.reshape(B).astype(jnp.int32)
    arowT = _sc_gather_rows(
        adj.transpose(1, 2, 0).reshape(_N * _N * B), aid_flat)
    actionsT, alpT, nsreT, nsimT = _tc_call(
        obs.T, node_obs.transpose(1, 2, 0), aid_flat.reshape(1, B), arowT,
        disturbances.T, ssm_state_re.T, ssm_state_im.T, masks.T,
        W1, W2, Wd, Wd2, Wm1, bm1.reshape(1, _HID), Wm2,
        bm2.reshape(1, _HID), Wa, ba.reshape(1, _ACT),
        log_std.reshape(1, _ACT), lam_re.reshape(1, _HID),
        lam_im.reshape(1, _HID), B_re, B_im, C_re, C_im, D, Wsm, Wso,
        bso.reshape(1, 1),
    )
    return (actionsT.T, alpT.T, rnn_states, nsreT.T, nsimT.T)


# speedup vs baseline: 4.2176x; 1.3907x over previous
import functools

import jax
import jax.numpy as jnp
import numpy as np
from jax import lax
from jax.experimental import pallas as pl
from jax.experimental.pallas import tpu as pltpu
from jax.experimental.pallas import tpu_sc as plsc

_N = 32
_F = 16
_OBS = 16
_HID = 64
_ACT = 2
_KP = 1.0
_MMAX = 1.0
_BB = 1024


def _sc_gather_rows(adj_flat, agent_id_flat):
    B = agent_id_flat.shape[0]
    info = plsc.get_sparse_core_info()
    nc, ns, L = info.num_cores, info.num_subcores, info.num_lanes
    nw = nc * ns
    bpw = B // nw
    nb = B * _N
    mesh = plsc.VectorSubcoreMesh(core_axis_name="c", subcore_axis_name="s")

    @functools.partial(
        pl.kernel,
        mesh=mesh,
        out_type=jax.ShapeDtypeStruct((_N, B), jnp.float32),
        scratch_types=[
            pltpu.VMEM((bpw,), jnp.int32),
            pltpu.VMEM((_N, bpw), jnp.int32),
            pltpu.VMEM((_N, bpw), jnp.float32),
            pltpu.SemaphoreType.DMA,
        ],
    )
    def k(adj_hbm, aid_hbm, arow_out, aid_v, idx_v, row_v, sem):
        wid = lax.axis_index("s") * nc + lax.axis_index("c")
        base = wid * bpw
        pltpu.sync_copy(aid_hbm.at[pl.ds(base, bpw)], aid_v)
        lane = lax.iota(jnp.int32, L)
        for j in range(_N):
            for i in range(bpw // L):
                ids = aid_v[pl.ds(i * L, L)]
                r = ids * _N + j
                idx_v[j, pl.ds(i * L, L)] = (
                    lax.shift_left(lax.shift_right_logical(r, 3), 3 + 7)
                    * (B // 128)
                    + (base // 128) * 1024
                    + lax.shift_left(r & 7, 7) + i * L + lane)
        copies = [
            pltpu.async_copy(adj_hbm.at[idx_v.at[j]], row_v.at[j], sem)
            for j in range(_N)
        ]
        for c in copies:
            c.wait()
        for j in range(_N):
            pltpu.sync_copy(row_v.at[j], arow_out.at[j, pl.ds(base, bpw)])

    return k(adj_flat, agent_id_flat)


def _tc_body(obs_r, nobs_r, aid_r, arow_r, dist_r, sre_r, sim_r, msk_r,
             W1_r, W2t_r, Wd_r, Wd2t_r, Wm1t_r, bm1_r, Wm2_r, bm2_r, Wat_r,
             ba_r, lstd_r, lre_r, lim_r, Bre_r, Bim_r, Cre_r, Cim_r, D_r,
             Wsm_r, Wsot_r, bso_r, act_out, alp_out, nsre_out, nsim_out):
    f32 = jnp.float32

    def dotT(w, x):
        return lax.dot_general(w, x, (((0,), (0,)), ((), ())),
                               preferred_element_type=f32)

    def dotN(wt, x):
        return lax.dot_general(wt, x, (((1,), (0,)), ((), ())),
                               preferred_element_type=f32)

    def bf(a):
        return a.astype(jnp.bfloat16).astype(f32)

    def col(v_r):
        return v_r[...].T

    aid = aid_r[...]
    oh = (lax.broadcasted_iota(jnp.int32, (_N, 1), 0) == aid).astype(f32)
    arow = arow_r[...]
    arow_q = bf(arow)
    rs = jnp.sum(arow_q, axis=0, keepdims=True)

    W1 = W1_r[...]
    agg = jnp.zeros((_HID, _BB), f32)
    hag = jnp.zeros((_HID, _BB), f32)
    for j in range(_N):
        h_j = jax.nn.relu(dotT(W1, nobs_r[j]))
        agg += arow_q[j:j + 1, :] * bf(h_j)
        hag += oh[j:j + 1, :] * h_j

    W2t = W2t_r[...]
    nbd = jax.nn.relu(dotN(W2t[:, :_HID], hag) + dotN(W2t[:, _HID:], agg))
    obs = obs_r[...]
    Wm1t = Wm1t_r[...]
    x = jax.nn.relu(dotN(Wm1t[:, :_OBS], obs) + dotN(Wm1t[:, _OBS:], nbd)
                    + col(bm1_r))
    actor = jax.nn.relu(dotT(Wm2_r[...], x) + col(bm2_r))
    mean = dotN(Wat_r[...], actor) + col(ba_r)
    u_gnn = jnp.tanh(mean)

    dn = jax.nn.relu(dotT(Wd_r[...], dist_r[...]))
    Wd2t = Wd2t_r[...]
    mag_g = jax.nn.relu(dotN(Wd2t[:, :_HID], dn)
                        + dotN(Wd2t[:, _HID:], rs * bf(dn)))

    reset = msk_r[...] == 0.0
    s_re = jnp.where(reset, 0.0, sre_r[...])
    s_im = jnp.where(reset, 0.0, sim_r[...])
    lre = col(lre_r)
    lim = col(lim_r)
    ns_re = lre * s_re - lim * s_im + dotT(Bre_r[...], mag_g)
    ns_im = lre * s_im + lim * s_re + dotT(Bim_r[...], mag_g)
    y_lru = (dotT(Cre_r[...], ns_re) - dotT(Cim_r[...], ns_im)
             + dotT(D_r[...], mag_g))
    ssm_raw = dotN(Wsot_r[...], jax.nn.relu(dotT(Wsm_r[...], y_lru))) \
        + col(bso_r)
    magnitude = jnp.clip(jax.nn.relu(ssm_raw), 1e-6, _MMAX)

    actions = obs[4:6, :] * _KP + magnitude * u_gnn
    lp = jnp.sum(-lstd_r[...] - 0.5 * np.float32(np.log(2.0 * np.pi)))
    ljt = jnp.sum(jnp.log(1.0 - u_gnn * u_gnn + 1e-8), axis=0, keepdims=True)
    ljm = jnp.log(magnitude + 1e-8) * float(_ACT)
    act_out[...] = actions
    alp_out[...] = lp - ljm - ljt
    nsre_out[...] = ns_re
    nsim_out[...] = ns_im


def _tc_call(obsT, nobsT, aidT, arowT, distT, sreT, simT, mskT,
             W1, W2, Wd, Wd2, Wm1, bm1, Wm2, bm2, Wa, ba, lstd, lre, lim,
             Bre, Bim, Cre, Cim, D, Wsm, Wso, bso):
    B = obsT.shape[1]
    grid = (B // _BB,)

    def bspec(shape):
        return pl.BlockSpec(shape[:-1] + (_BB,),
                            lambda i: (0,) * (len(shape) - 1) + (i,))

    def wspec(shape):
        return pl.BlockSpec(shape, lambda i: (0,) * len(shape))

    batch_args = [obsT, nobsT, aidT, arowT, distT, sreT, simT, mskT]
    weight_args = [W1, W2, Wd, Wd2, Wm1, bm1, Wm2, bm2, Wa, ba, lstd, lre,
                   lim, Bre, Bim, Cre, Cim, D, Wsm, Wso, bso]
    in_specs = [bspec(a.shape) for a in batch_args] + \
               [wspec(a.shape) for a in weight_args]
    out_shapes = [
        jax.ShapeDtypeStruct((_ACT, B), jnp.float32),
        jax.ShapeDtypeStruct((1, B), jnp.float32),
        jax.ShapeDtypeStruct((_HID, B), jnp.float32),
        jax.ShapeDtypeStruct((_HID, B), jnp.float32),
    ]
    out_specs = [bspec(s.shape) for s in out_shapes]
    return pl.pallas_call(
        _tc_body,
        grid=grid,
        in_specs=in_specs,
        out_specs=out_specs,
        out_shape=out_shapes,
    )(*batch_args, *weight_args)


def kernel(obs, node_obs, adj, agent_id, rnn_states, ssm_state_re,
           ssm_state_im, disturbances, masks, W1, W2, Wd, Wd2, Wm1, bm1, Wm2,
           bm2, Wa, ba, log_std, lam_re, lam_im, B_re, B_im, C_re, C_im, D,
           Wsm, Wso, bso):
    B = obs.shape[0]
    aid_flat = agent_id.reshape(B).astype(jnp.int32)
    adj_flat = (adj.transpose(1, 2, 0)
                .reshape(_N * _N // 8, 8, B // 128, 128)
                .transpose(0, 2, 1, 3)
                .reshape(_N * _N * B))
    arowT = _sc_gather_rows(adj_flat, aid_flat)
    actionsT, alpT, nsreT, nsimT = _tc_call(
        obs.T, node_obs.transpose(1, 2, 0), aid_flat.reshape(1, B), arowT,
        disturbances.T, ssm_state_re.T, ssm_state_im.T, masks.T,
        W1, W2.T, Wd, Wd2.T, Wm1.T, bm1.reshape(1, _HID), Wm2,
        bm2.reshape(1, _HID), Wa.T, ba.reshape(1, _ACT),
        log_std.reshape(1, _ACT), lam_re.reshape(1, _HID),
        lam_im.reshape(1, _HID), B_re, B_im, C_re, C_im, D, Wsm, Wso.T,
        bso.reshape(1, 1),
    )
    return (actionsT.T, alpT.T, rnn_states, nsreT.T, nsimT.T)


# speedup vs baseline: 4.4628x; 1.0581x over previous
import functools

import jax
import jax.numpy as jnp
import numpy as np
from jax import lax
from jax.experimental import pallas as pl
from jax.experimental.pallas import tpu as pltpu
from jax.experimental.pallas import tpu_sc as plsc

_N = 32
_F = 16
_OBS = 16
_HID = 64
_ACT = 2
_KP = 1.0
_MMAX = 1.0
_BB = 2048


def _sc_gather_rows(adj_flat, agent_id_flat):
    B = agent_id_flat.shape[0]
    info = plsc.get_sparse_core_info()
    nc, ns, L = info.num_cores, info.num_subcores, info.num_lanes
    nw = nc * ns
    bpw = B // nw
    nb = B * _N
    mesh = plsc.VectorSubcoreMesh(core_axis_name="c", subcore_axis_name="s")

    @functools.partial(
        pl.kernel,
        mesh=mesh,
        out_type=jax.ShapeDtypeStruct((_N, B), jnp.float32),
        scratch_types=[
            pltpu.VMEM((bpw,), jnp.int32),
            pltpu.VMEM((_N, bpw), jnp.int32),
            pltpu.VMEM((_N, bpw), jnp.float32),
            pltpu.SemaphoreType.DMA,
        ],
    )
    def k(adj_hbm, aid_hbm, arow_out, aid_v, idx_v, row_v, sem):
        wid = lax.axis_index("s") * nc + lax.axis_index("c")
        base = wid * bpw
        pltpu.sync_copy(aid_hbm.at[pl.ds(base, bpw)], aid_v)
        lane = lax.iota(jnp.int32, L)
        for i in range(bpw // L):
            ids = aid_v[pl.ds(i * L, L)]
            bvec = ids * (_N * B) + ((base // 128) * 1024 + i * L) + lane
            for j in range(_N):
                idx_v[j, pl.ds(i * L, L)] = bvec + (
                    (j // 8) * (8 * B) + (j % 8) * 128)
        copies = [
            pltpu.async_copy(adj_hbm.at[idx_v.at[j]], row_v.at[j], sem)
            for j in range(_N)
        ]
        for c in copies:
            c.wait()
        for j in range(_N):
            pltpu.sync_copy(row_v.at[j], arow_out.at[j, pl.ds(base, bpw)])

    return k(adj_flat, agent_id_flat)


def _tc_body(obs_r, nobs_r, aid_r, arow_r, dist_r, sre_r, sim_r, msk_r,
             W1_r, W2t_r, Wd_r, Wd2t_r, Wm1t_r, bm1_r, Wm2_r, bm2_r, Wat_r,
             ba_r, lstd_r, lre_r, lim_r, Bre_r, Bim_r, Cre_r, Cim_r, D_r,
             Wsm_r, Wsot_r, bso_r, act_out, alp_out, nsre_out, nsim_out):
    f32 = jnp.float32

    def dotT(w, x):
        return lax.dot_general(w, x, (((0,), (0,)), ((), ())),
                               preferred_element_type=f32)

    def dotN(wt, x):
        return lax.dot_general(wt, x, (((1,), (0,)), ((), ())),
                               preferred_element_type=f32)

    def bf(a):
        return a.astype(jnp.bfloat16).astype(f32)

    def col(v_r):
        return v_r[...].T

    aid = aid_r[...]
    oh = (lax.broadcasted_iota(jnp.int32, (_N, 1), 0) == aid).astype(f32)
    arow = arow_r[...]
    arow_q = bf(arow)
    rs = jnp.sum(arow_q, axis=0, keepdims=True)

    W1 = W1_r[...]
    agg = jnp.zeros((_HID, _BB), f32)
    hag = jnp.zeros((_HID, _BB), f32)
    for j in range(_N):
        h_j = jax.nn.relu(dotT(W1, nobs_r[j]))
        agg += arow_q[j:j + 1, :] * bf(h_j)
        hag += oh[j:j + 1, :] * h_j

    W2t = W2t_r[...]
    nbd = jax.nn.relu(dotN(W2t[:, :_HID], hag) + dotN(W2t[:, _HID:], agg))
    obs = obs_r[...]
    Wm1t = Wm1t_r[...]
    x = jax.nn.relu(dotN(Wm1t[:, :_OBS], obs) + dotN(Wm1t[:, _OBS:], nbd)
                    + col(bm1_r))
    actor = jax.nn.relu(dotT(Wm2_r[...], x) + col(bm2_r))
    mean = dotN(Wat_r[...], actor) + col(ba_r)
    u_gnn = jnp.tanh(mean)

    dn = jax.nn.relu(dotT(Wd_r[...], dist_r[...]))
    Wd2t = Wd2t_r[...]
    mag_g = jax.nn.relu(dotN(Wd2t[:, :_HID], dn)
                        + dotN(Wd2t[:, _HID:], rs * bf(dn)))

    reset = msk_r[...] == 0.0
    s_re = jnp.where(reset, 0.0, sre_r[...])
    s_im = jnp.where(reset, 0.0, sim_r[...])
    lre = col(lre_r)
    lim = col(lim_r)
    ns_re = lre * s_re - lim * s_im + dotT(Bre_r[...], mag_g)
    ns_im = lre * s_im + lim * s_re + dotT(Bim_r[...], mag_g)
    y_lru = (dotT(Cre_r[...], ns_re) - dotT(Cim_r[...], ns_im)
             + dotT(D_r[...], mag_g))
    ssm_raw = dotN(Wsot_r[...], jax.nn.relu(dotT(Wsm_r[...], y_lru))) \
        + col(bso_r)
    magnitude = jnp.clip(jax.nn.relu(ssm_raw), 1e-6, _MMAX)

    actions = obs[4:6, :] * _KP + magnitude * u_gnn
    lp = jnp.sum(-lstd_r[...] - 0.5 * np.float32(np.log(2.0 * np.pi)))
    ljt = jnp.sum(jnp.log(1.0 - u_gnn * u_gnn + 1e-8), axis=0, keepdims=True)
    ljm = jnp.log(magnitude + 1e-8) * float(_ACT)
    act_out[...] = actions
    alp_out[...] = lp - ljm - ljt
    nsre_out[...] = ns_re
    nsim_out[...] = ns_im


def _tc_call(obsT, nobsT, aidT, arowT, distT, sreT, simT, mskT,
             W1, W2, Wd, Wd2, Wm1, bm1, Wm2, bm2, Wa, ba, lstd, lre, lim,
             Bre, Bim, Cre, Cim, D, Wsm, Wso, bso):
    B = obsT.shape[1]
    grid = (B // _BB,)

    def bspec(shape):
        return pl.BlockSpec(shape[:-1] + (_BB,),
                            lambda i: (0,) * (len(shape) - 1) + (i,))

    def wspec(shape):
        return pl.BlockSpec(shape, lambda i: (0,) * len(shape))

    batch_args = [obsT, nobsT, aidT, arowT, distT, sreT, simT, mskT]
    weight_args = [W1, W2, Wd, Wd2, Wm1, bm1, Wm2, bm2, Wa, ba, lstd, lre,
                   lim, Bre, Bim, Cre, Cim, D, Wsm, Wso, bso]
    in_specs = [bspec(a.shape) for a in batch_args] + \
               [wspec(a.shape) for a in weight_args]
    out_shapes = [
        jax.ShapeDtypeStruct((_ACT, B), jnp.float32),
        jax.ShapeDtypeStruct((1, B), jnp.float32),
        jax.ShapeDtypeStruct((_HID, B), jnp.float32),
        jax.ShapeDtypeStruct((_HID, B), jnp.float32),
    ]
    out_specs = [bspec(s.shape) for s in out_shapes]
    return pl.pallas_call(
        _tc_body,
        grid=grid,
        in_specs=in_specs,
        out_specs=out_specs,
        out_shape=out_shapes,
    )(*batch_args, *weight_args)


def kernel(obs, node_obs, adj, agent_id, rnn_states, ssm_state_re,
           ssm_state_im, disturbances, masks, W1, W2, Wd, Wd2, Wm1, bm1, Wm2,
           bm2, Wa, ba, log_std, lam_re, lam_im, B_re, B_im, C_re, C_im, D,
           Wsm, Wso, bso):
    B = obs.shape[0]
    aid_flat = agent_id.reshape(B).astype(jnp.int32)
    adj_flat = (adj.transpose(1, 2, 0)
                .reshape(_N * _N // 8, 8, B // 128, 128)
                .transpose(0, 2, 1, 3)
                .reshape(_N * _N * B))
    arowT = _sc_gather_rows(adj_flat, aid_flat)
    actionsT, alpT, nsreT, nsimT = _tc_call(
        obs.T, node_obs.transpose(1, 2, 0), aid_flat.reshape(1, B), arowT,
        disturbances.T, ssm_state_re.T, ssm_state_im.T, masks.T,
        W1, W2.T, Wd, Wd2.T, Wm1.T, bm1.reshape(1, _HID), Wm2,
        bm2.reshape(1, _HID), Wa.T, ba.reshape(1, _ACT),
        log_std.reshape(1, _ACT), lam_re.reshape(1, _HID),
        lam_im.reshape(1, _HID), B_re, B_im, C_re, C_im, D, Wsm, Wso.T,
        bso.reshape(1, 1),
    )
    return (actionsT.T, alpT.T, rnn_states, nsreT.T, nsimT.T)


# speedup vs baseline: 4.4669x; 1.0009x over previous
import functools

import jax
import jax.numpy as jnp
import numpy as np
from jax import lax
from jax.experimental import pallas as pl
from jax.experimental.pallas import tpu as pltpu
from jax.experimental.pallas import tpu_sc as plsc

_N = 32
_F = 16
_OBS = 16
_HID = 64
_ACT = 2
_KP = 1.0
_MMAX = 1.0
_BB = 2048


def _sc_gather_rows(adj_flat, agent_id_flat):
    B = agent_id_flat.shape[0]
    info = plsc.get_sparse_core_info()
    nc, ns, L = info.num_cores, info.num_subcores, info.num_lanes
    nw = nc * ns
    bpw = B // nw
    mesh = plsc.VectorSubcoreMesh(core_axis_name="c", subcore_axis_name="s")

    @functools.partial(
        pl.kernel,
        mesh=mesh,
        out_type=jax.ShapeDtypeStruct((_N, B), jnp.float32),
        scratch_types=[
            pltpu.VMEM((bpw,), jnp.int32),
            pltpu.VMEM((_N, bpw), jnp.int32),
            pltpu.VMEM((_N, bpw), jnp.float32),
            pltpu.SemaphoreType.DMA,
        ],
    )
    def k(adj_hbm, aid_hbm, arow_out, aid_v, idx_v, row_v, sem):
        wid = lax.axis_index("s") * nc + lax.axis_index("c")
        base = wid * bpw
        pltpu.sync_copy(aid_hbm.at[pl.ds(base, bpw)], aid_v)
        lane = lax.iota(jnp.int32, L)
        for i in range(bpw // L):
            ids = aid_v[pl.ds(i * L, L)]
            bvec = ids * (_N * B) + ((base // 128) * 1024 + i * L) + lane
            for j in range(_N):
                idx_v[j, pl.ds(i * L, L)] = bvec + (
                    (j // 8) * (8 * B) + (j % 8) * 128)
        copies = [
            pltpu.async_copy(adj_hbm.at[idx_v.at[j]], row_v.at[j], sem)
            for j in range(_N)
        ]
        for c in copies:
            c.wait()
        for j in range(_N):
            pltpu.sync_copy(row_v.at[j], arow_out.at[j, pl.ds(base, bpw)])

    return k(adj_flat, agent_id_flat)


def _tc_body(obs_r, nobs_r, aid_r, arow_r, dist_r, sre_r, sim_r, msk_r,
             W1_r, W2t_r, Wd_r, Wd2t_r, Wm1t_r, bm1_r, Wm2_r, bm2_r, Wat_r,
             ba_r, lstd_r, lre_r, lim_r, Bre_r, Bim_r, Cre_r, Cim_r, D_r,
             Wsm_r, Wsot_r, bso_r, act_out, alp_out, nsre_out, nsim_out):
    f32 = jnp.float32

    def dotT(w, x):
        return lax.dot_general(w, x, (((0,), (0,)), ((), ())),
                               preferred_element_type=f32)

    def dotN(wt, x):
        return lax.dot_general(wt, x, (((1,), (0,)), ((), ())),
                               preferred_element_type=f32)

    def bf(a):
        return a.astype(jnp.bfloat16).astype(f32)

    def col(v_r):
        return v_r[...].T

    aid = aid_r[...]
    oh = (lax.broadcasted_iota(jnp.int32, (_N, 1), 0) == aid).astype(f32)
    arow = arow_r[...]
    arow_q = bf(arow)
    rs = jnp.sum(arow_q, axis=0, keepdims=True)

    W1 = W1_r[...]
    agg = jnp.zeros((_HID, _BB), f32)
    hag = jnp.zeros((_HID, _BB), f32)
    for j in range(_N):
        h_j = jax.nn.relu(dotT(W1, nobs_r[j]))
        agg += arow_q[j:j + 1, :] * bf(h_j)
        hag += oh[j:j + 1, :] * h_j

    W2t = W2t_r[...]
    nbd = jax.nn.relu(dotN(W2t[:, :_HID], hag) + dotN(W2t[:, _HID:], agg))
    obs = obs_r[...]
    Wm1t = Wm1t_r[...]
    x = jax.nn.relu(dotN(Wm1t[:, :_OBS], obs) + dotN(Wm1t[:, _OBS:], nbd)
                    + col(bm1_r))
    actor = jax.nn.relu(dotT(Wm2_r[...], x) + col(bm2_r))
    mean = dotN(Wat_r[...], actor) + col(ba_r)
    u_gnn = jnp.tanh(mean)

    dn = jax.nn.relu(dotT(Wd_r[...], dist_r[...]))
    Wd2t = Wd2t_r[...]
    mag_g = jax.nn.relu(dotN(Wd2t[:, :_HID], dn)
                        + dotN(Wd2t[:, _HID:], rs * bf(dn)))

    reset = msk_r[...] == 0.0
    s_re = jnp.where(reset, 0.0, sre_r[...])
    s_im = jnp.where(reset, 0.0, sim_r[...])
    lre = col(lre_r)
    lim = col(lim_r)
    ns_re = lre * s_re - lim * s_im + dotT(Bre_r[...], mag_g)
    ns_im = lre * s_im + lim * s_re + dotT(Bim_r[...], mag_g)
    y_lru = (dotT(Cre_r[...], ns_re) - dotT(Cim_r[...], ns_im)
             + dotT(D_r[...], mag_g))
    ssm_raw = dotN(Wsot_r[...], jax.nn.relu(dotT(Wsm_r[...], y_lru))) \
        + col(bso_r)
    magnitude = jnp.clip(jax.nn.relu(ssm_raw), 1e-6, _MMAX)

    actions = obs[4:6, :] * _KP + magnitude * u_gnn
    lp = jnp.sum(-lstd_r[...] - 0.5 * np.float32(np.log(2.0 * np.pi)))
    ljt = jnp.sum(jnp.log(1.0 - u_gnn * u_gnn + 1e-8), axis=0, keepdims=True)
    ljm = jnp.log(magnitude + 1e-8) * float(_ACT)
    act_out[...] = actions
    alp_out[...] = lp - ljm - ljt
    nsre_out[...] = ns_re
    nsim_out[...] = ns_im


def _tc_call(obsT, nobsT, aidT, arowT, distT, sreT, simT, mskT,
             W1, W2, Wd, Wd2, Wm1, bm1, Wm2, bm2, Wa, ba, lstd, lre, lim,
             Bre, Bim, Cre, Cim, D, Wsm, Wso, bso):
    B = obsT.shape[1]
    grid = (B // _BB,)

    def bspec(shape):
        return pl.BlockSpec(shape[:-1] + (_BB,),
                            lambda i: (0,) * (len(shape) - 1) + (i,))

    def wspec(shape):
        return pl.BlockSpec(shape, lambda i: (0,) * len(shape))

    batch_args = [obsT, nobsT, aidT, arowT, distT, sreT, simT, mskT]
    weight_args = [W1, W2, Wd, Wd2, Wm1, bm1, Wm2, bm2, Wa, ba, lstd, lre,
                   lim, Bre, Bim, Cre, Cim, D, Wsm, Wso, bso]
    in_specs = [bspec(a.shape) for a in batch_args] + \
               [wspec(a.shape) for a in weight_args]
    out_shapes = [
        jax.ShapeDtypeStruct((_ACT, B), jnp.float32),
        jax.ShapeDtypeStruct((1, B), jnp.float32),
        jax.ShapeDtypeStruct((_HID, B), jnp.float32),
        jax.ShapeDtypeStruct((_HID, B), jnp.float32),
    ]
    out_specs = [bspec(s.shape) for s in out_shapes]
    return pl.pallas_call(
        _tc_body,
        grid=grid,
        in_specs=in_specs,
        out_specs=out_specs,
        out_shape=out_shapes,
    )(*batch_args, *weight_args)


def kernel(obs, node_obs, adj, agent_id, rnn_states, ssm_state_re,
           ssm_state_im, disturbances, masks, W1, W2, Wd, Wd2, Wm1, bm1, Wm2,
           bm2, Wa, ba, log_std, lam_re, lam_im, B_re, B_im, C_re, C_im, D,
           Wsm, Wso, bso):
    B = obs.shape[0]
    aid_flat = agent_id.reshape(B).astype(jnp.int32)
    adj_flat = (adj.transpose(1, 2, 0)
                .reshape(_N * _N // 8, 8, B // 128, 128)
                .transpose(0, 2, 1, 3)
                .reshape(_N * _N * B))
    arowT = _sc_gather_rows(adj_flat, aid_flat)
    actionsT, alpT, nsreT, nsimT = _tc_call(
        obs.T, node_obs.transpose(1, 2, 0), aid_flat.reshape(1, B), arowT,
        disturbances.T, ssm_state_re.T, ssm_state_im.T, masks.T,
        W1, W2.T, Wd, Wd2.T, Wm1.T, bm1.reshape(1, _HID), Wm2,
        bm2.reshape(1, _HID), Wa.T, ba.reshape(1, _ACT),
        log_std.reshape(1, _ACT), lam_re.reshape(1, _HID),
        lam_im.reshape(1, _HID), B_re, B_im, C_re, C_im, D, Wsm, Wso.T,
        bso.reshape(1, 1),
    )
    return (actionsT.T, alpT.T, rnn_states, nsreT.T, nsimT.T)


# speedup vs baseline: 4.5644x; 1.0218x over previous
import functools

import jax
import jax.numpy as jnp
import numpy as np
from jax import lax
from jax.experimental import pallas as pl
from jax.experimental.pallas import tpu as pltpu
from jax.experimental.pallas import tpu_sc as plsc

_N = 32
_F = 16
_OBS = 16
_HID = 64
_ACT = 2
_KP = 1.0
_MMAX = 1.0
_BB = 4096


def _sc_gather_rows(adj_flat, agent_id_flat):
    B = agent_id_flat.shape[0]
    info = plsc.get_sparse_core_info()
    nc, ns, L = info.num_cores, info.num_subcores, info.num_lanes
    nw = nc * ns
    bpw = B // nw
    mesh = plsc.VectorSubcoreMesh(core_axis_name="c", subcore_axis_name="s")

    @functools.partial(
        pl.kernel,
        mesh=mesh,
        out_type=jax.ShapeDtypeStruct((_N, B), jnp.float32),
        scratch_types=[
            pltpu.VMEM((bpw,), jnp.int32),
            pltpu.VMEM((_N, bpw), jnp.int32),
            pltpu.VMEM((_N, bpw), jnp.float32),
            pltpu.SemaphoreType.DMA,
        ],
    )
    def k(adj_hbm, aid_hbm, arow_out, aid_v, idx_v, row_v, sem):
        wid = lax.axis_index("s") * nc + lax.axis_index("c")
        base = wid * bpw
        pltpu.sync_copy(aid_hbm.at[pl.ds(base, bpw)], aid_v)
        lane = lax.iota(jnp.int32, L)
        for i in range(bpw // L):
            ids = aid_v[pl.ds(i * L, L)]
            bvec = ids * (_N * B) + ((base // 128) * 1024 + i * L) + lane
            for j in range(_N):
                idx_v[j, pl.ds(i * L, L)] = bvec + (
                    (j // 8) * (8 * B) + (j % 8) * 128)
        copies = [
            pltpu.async_copy(adj_hbm.at[idx_v.at[j]], row_v.at[j], sem)
            for j in range(_N)
        ]
        for c in copies:
            c.wait()
        for j in range(_N):
            pltpu.sync_copy(row_v.at[j], arow_out.at[j, pl.ds(base, bpw)])

    return k(adj_flat, agent_id_flat)


def _tc_body(obs_r, nobs_r, aid_r, arow_r, dist_r, sre_r, sim_r, msk_r,
             W1_r, W2t_r, Wd_r, Wd2t_r, Wm1t_r, bm1_r, Wm2_r, bm2_r, Wat_r,
             ba_r, lstd_r, lre_r, lim_r, Bre_r, Bim_r, Cre_r, Cim_r, D_r,
             Wsm_r, Wsot_r, bso_r, act_out, alp_out, nsre_out, nsim_out):
    f32 = jnp.float32

    def dotT(w, x):
        return lax.dot_general(w, x, (((0,), (0,)), ((), ())),
                               preferred_element_type=f32)

    def dotN(wt, x):
        return lax.dot_general(wt, x, (((1,), (0,)), ((), ())),
                               preferred_element_type=f32)

    def bf(a):
        return a.astype(jnp.bfloat16).astype(f32)

    def col(v_r):
        return v_r[...].T

    aid = aid_r[...]
    oh = (lax.broadcasted_iota(jnp.int32, (_N, 1), 0) == aid).astype(f32)
    arow = arow_r[...]
    arow_q = bf(arow)
    rs = jnp.sum(arow_q, axis=0, keepdims=True)

    W1 = W1_r[...]
    agg = jnp.zeros((_HID, _BB), f32)
    hag = jnp.zeros((_HID, _BB), f32)
    for j in range(_N):
        h_j = jax.nn.relu(dotT(W1, nobs_r[j]))
        agg += arow_q[j:j + 1, :] * bf(h_j)
        hag += oh[j:j + 1, :] * h_j

    W2t = W2t_r[...]
    nbd = jax.nn.relu(dotN(W2t[:, :_HID], hag) + dotN(W2t[:, _HID:], agg))
    obs = obs_r[...]
    Wm1t = Wm1t_r[...]
    x = jax.nn.relu(dotN(Wm1t[:, :_OBS], obs) + dotN(Wm1t[:, _OBS:], nbd)
                    + col(bm1_r))
    actor = jax.nn.relu(dotT(Wm2_r[...], x) + col(bm2_r))
    mean = dotN(Wat_r[...], actor) + col(ba_r)
    u_gnn = jnp.tanh(mean)

    dn = jax.nn.relu(dotT(Wd_r[...], dist_r[...]))
    Wd2t = Wd2t_r[...]
    mag_g = jax.nn.relu(dotN(Wd2t[:, :_HID], dn)
                        + dotN(Wd2t[:, _HID:], rs * bf(dn)))

    reset = msk_r[...] == 0.0
    s_re = jnp.where(reset, 0.0, sre_r[...])
    s_im = jnp.where(reset, 0.0, sim_r[...])
    lre = col(lre_r)
    lim = col(lim_r)
    ns_re = lre * s_re - lim * s_im + dotT(Bre_r[...], mag_g)
    ns_im = lre * s_im + lim * s_re + dotT(Bim_r[...], mag_g)
    y_lru = (dotT(Cre_r[...], ns_re) - dotT(Cim_r[...], ns_im)
             + dotT(D_r[...], mag_g))
    ssm_raw = dotN(Wsot_r[...], jax.nn.relu(dotT(Wsm_r[...], y_lru))) \
        + col(bso_r)
    magnitude = jnp.clip(jax.nn.relu(ssm_raw), 1e-6, _MMAX)

    actions = obs[4:6, :] * _KP + magnitude * u_gnn
    lp = jnp.sum(-lstd_r[...] - 0.5 * np.float32(np.log(2.0 * np.pi)))
    ljt = jnp.sum(jnp.log(1.0 - u_gnn * u_gnn + 1e-8), axis=0, keepdims=True)
    ljm = jnp.log(magnitude + 1e-8) * float(_ACT)
    act_out[...] = actions
    alp_out[...] = lp - ljm - ljt
    nsre_out[...] = ns_re
    nsim_out[...] = ns_im


def _tc_call(obsT, nobsT, aidT, arowT, distT, sreT, simT, mskT,
             W1, W2, Wd, Wd2, Wm1, bm1, Wm2, bm2, Wa, ba, lstd, lre, lim,
             Bre, Bim, Cre, Cim, D, Wsm, Wso, bso):
    B = obsT.shape[1]
    grid = (B // _BB,)

    def bspec(shape):
        return pl.BlockSpec(shape[:-1] + (_BB,),
                            lambda i: (0,) * (len(shape) - 1) + (i,))

    def wspec(shape):
        return pl.BlockSpec(shape, lambda i: (0,) * len(shape))

    batch_args = [obsT, nobsT, aidT, arowT, distT, sreT, simT, mskT]
    weight_args = [W1, W2, Wd, Wd2, Wm1, bm1, Wm2, bm2, Wa, ba, lstd, lre,
                   lim, Bre, Bim, Cre, Cim, D, Wsm, Wso, bso]
    in_specs = [bspec(a.shape) for a in batch_args] + \
               [wspec(a.shape) for a in weight_args]
    out_shapes = [
        jax.ShapeDtypeStruct((_ACT, B), jnp.float32),
        jax.ShapeDtypeStruct((1, B), jnp.float32),
        jax.ShapeDtypeStruct((_HID, B), jnp.float32),
        jax.ShapeDtypeStruct((_HID, B), jnp.float32),
    ]
    out_specs = [bspec(s.shape) for s in out_shapes]
    return pl.pallas_call(
        _tc_body,
        grid=grid,
        in_specs=in_specs,
        out_specs=out_specs,
        out_shape=out_shapes,
    )(*batch_args, *weight_args)


def kernel(obs, node_obs, adj, agent_id, rnn_states, ssm_state_re,
           ssm_state_im, disturbances, masks, W1, W2, Wd, Wd2, Wm1, bm1, Wm2,
           bm2, Wa, ba, log_std, lam_re, lam_im, B_re, B_im, C_re, C_im, D,
           Wsm, Wso, bso):
    B = obs.shape[0]
    aid_flat = agent_id.reshape(B).astype(jnp.int32)
    adj_flat = (adj.transpose(1, 2, 0)
                .reshape(_N * _N // 8, 8, B // 128, 128)
                .transpose(0, 2, 1, 3)
                .reshape(_N * _N * B))
    arowT = _sc_gather_rows(adj_flat, aid_flat)
    actionsT, alpT, nsreT, nsimT = _tc_call(
        obs.T, node_obs.transpose(1, 2, 0), aid_flat.reshape(1, B), arowT,
        disturbances.T, ssm_state_re.T, ssm_state_im.T, masks.T,
        W1, W2.T, Wd, Wd2.T, Wm1.T, bm1.reshape(1, _HID), Wm2,
        bm2.reshape(1, _HID), Wa.T, ba.reshape(1, _ACT),
        log_std.reshape(1, _ACT), lam_re.reshape(1, _HID),
        lam_im.reshape(1, _HID), B_re, B_im, C_re, C_im, D, Wsm, Wso.T,
        bso.reshape(1, 1),
    )
    return (actionsT.T, alpT.T, rnn_states, nsreT.T, nsimT.T)
